# Initial kernel scaffold; baseline (speedup 1.0000x reference)
import jax
import jax.numpy as jnp
from jax.experimental import pallas as pl


def kernel(x, edge_attr, edge_sh, emb_w1, emb_b1, emb_w2, emb_b2, gate_w1, gate_b1, gate_w2, gate_b2, W1, W2, edge_index):
    raise NotImplementedError("write your pallas kernel here")


# speedup vs baseline: 224.8097x; 224.8097x over previous
import functools

import jax
import jax.numpy as jnp
from jax import lax
from jax.experimental import pallas as pl
from jax.experimental.pallas import tpu as pltpu
from jax.experimental.pallas import tpu_sc as plsc

_NS = 16
_N = 10000
_E = 160000
_D = 128
_SH = 9
_DE = 64

_CH = 128
_NCH = _E // _CH
_NW = 32
_CPT = _NCH // _NW
_REM = _NCH - _CPT * _NW
_RPS = 624
_RTAIL = _N - 16 * _RPS

_TS = _D + 48
_TSP = 256
_TDP = 128

_mesh = plsc.VectorSubcoreMesh(core_axis_name="c", subcore_axis_name="s")


@functools.partial(
    pl.kernel,
    mesh=_mesh,
    out_type=[
        jax.ShapeDtypeStruct((_E, _TSP), jnp.float32),
        jax.ShapeDtypeStruct((_E, _TDP), jnp.float32),
    ],
    scratch_types=[
        pltpu.VMEM((1, _CH), jnp.int32),
        pltpu.VMEM((1, _CH), jnp.int32),
        pltpu.VMEM((_CH, _TSP), jnp.float32),
        pltpu.VMEM((_CH, _TDP), jnp.float32),
        pltpu.SemaphoreType.DMA,
        pltpu.SemaphoreType.DMA,
    ],
)
def _sc_gather(tsrc, tdst, src2, dst2, gsrc, gdst,
               idx_s, idx_d, buf_s, buf_d, sem_s, sem_d):
    wid = lax.axis_index("s") * 2 + lax.axis_index("c")
    c0 = wid * _CPT

    def do_chunk(ci):
        pltpu.sync_copy(src2.at[pl.ds(ci, 1)], idx_s)
        pltpu.sync_copy(dst2.at[pl.ds(ci, 1)], idx_d)
        cp_s = pltpu.async_copy(tsrc.at[idx_s.at[0]], buf_s, sem_s)
        cp_d = pltpu.async_copy(tdst.at[idx_d.at[0]], buf_d, sem_d)
        cp_s.wait()
        cp_d.wait()
        pltpu.sync_copy(buf_s, gsrc.at[pl.ds(ci * _CH, _CH)])
        pltpu.sync_copy(buf_d, gdst.at[pl.ds(ci * _CH, _CH)])

    def body(j, carry):
        do_chunk(c0 + j)
        return carry

    lax.fori_loop(0, _CPT, body, 0)

    @pl.when(wid < _REM)
    def _():
        do_chunk(_NW * _CPT + wid)


@functools.partial(
    pl.kernel,
    mesh=_mesh,
    out_type=jax.ShapeDtypeStruct((2, _N, _D), jnp.float32),
    scratch_types=[
        pltpu.VMEM((1, _CH), jnp.int32),
        pltpu.VMEM((_CH, _D), jnp.float32),
        pltpu.VMEM_SHARED((_N, _D), jnp.float32),
    ],
)
def _sc_scatter(msg, dst2, out, idx, buf, acc):
    cid = lax.axis_index("c")
    sid = lax.axis_index("s")
    wid = sid * 2 + cid

    def zrow(i, carry):
        for k in range(_D // 16):
            buf[i, pl.ds(k * 16, 16)] = jnp.zeros((16,), jnp.float32)
        return carry

    lax.fori_loop(0, _CH, zrow, 0)
    r0 = sid * _RPS
    for t in range(4):
        pltpu.sync_copy(buf, acc.at[pl.ds(r0 + t * _CH, _CH)])
    pltpu.sync_copy(buf.at[pl.ds(0, _RPS - 4 * _CH)],
                    acc.at[pl.ds(r0 + 4 * _CH, _RPS - 4 * _CH)])

    @pl.when(sid == 0)
    def _():
        pltpu.sync_copy(buf.at[pl.ds(0, _RTAIL)],
                        acc.at[pl.ds(16 * _RPS, _RTAIL)])

    plsc.subcore_barrier()

    def do_chunk(ci):
        pltpu.sync_copy(dst2.at[pl.ds(ci, 1)], idx)
        pltpu.sync_copy(msg.at[pl.ds(ci * _CH, _CH)], buf)
        pltpu.sync_copy(buf, acc.at[idx.at[0]], add=True)

    c0 = wid * _CPT

    def body(j, carry):
        do_chunk(c0 + j)
        return carry

    lax.fori_loop(0, _CPT, body, 0)

    @pl.when(wid < _REM)
    def _():
        do_chunk(_NW * _CPT + wid)

    plsc.subcore_barrier()
    pltpu.sync_copy(acc.at[pl.ds(r0, _RPS)], out.at[cid, pl.ds(r0, _RPS)])

    @pl.when(sid == 0)
    def _():
        pltpu.sync_copy(acc.at[pl.ds(16 * _RPS, _RTAIL)],
                        out.at[cid, pl.ds(16 * _RPS, _RTAIL)])


_BE = 2000
_BN = 1000


def _pre_body(ea_ref, esh_ref, ew1_ref, eb1_ref, ew2_ref, eb2_ref,
              gw1_ref, gb1_ref, w2_ref,
              pre0_ref, pre1_ref, shw0_ref, shw1_ref):
    e = jnp.maximum(
        jnp.dot(ea_ref[...], ew1_ref[...], preferred_element_type=jnp.float32)
        + eb1_ref[...], 0.0)
    e = jnp.dot(e, ew2_ref[...], preferred_element_type=jnp.float32) + eb2_ref[...]
    esh = esh_ref[...]
    for l, (pre_ref, shw_ref) in enumerate(((pre0_ref, shw0_ref),
                                            (pre1_ref, shw1_ref))):
        w_top = gw1_ref[l, :16, :]
        pre_ref[...] = (jnp.dot(e, w_top, preferred_element_type=jnp.float32)
                        + gb1_ref[l])
        shw_ref[...] = jnp.dot(esh, w2_ref[l],
                               preferred_element_type=jnp.float32)


def _tc_pre(edge_attr, edge_sh, ew1, eb1, ew2, eb2, gw1, gb1, w2):
    grid = (_E // _BE,)
    return pl.pallas_call(
        _pre_body,
        grid=grid,
        in_specs=[
            pl.BlockSpec((_BE, _DE), lambda i: (i, 0)),
            pl.BlockSpec((_BE, _SH), lambda i: (i, 0)),
            pl.BlockSpec((_DE, _NS), lambda i: (0, 0)),
            pl.BlockSpec((1, _NS), lambda i: (0, 0)),
            pl.BlockSpec((_NS, _NS), lambda i: (0, 0)),
            pl.BlockSpec((1, _NS), lambda i: (0, 0)),
            pl.BlockSpec((2, 48, 48), lambda i: (0, 0, 0)),
            pl.BlockSpec((2, 1, 48), lambda i: (0, 0, 0)),
            pl.BlockSpec((2, _SH, _D), lambda i: (0, 0, 0)),
        ],
        out_specs=[
            pl.BlockSpec((_BE, 48), lambda i: (i, 0)),
            pl.BlockSpec((_BE, 48), lambda i: (i, 0)),
            pl.BlockSpec((_BE, _D), lambda i: (i, 0)),
            pl.BlockSpec((_BE, _D), lambda i: (i, 0)),
        ],
        out_shape=[
            jax.ShapeDtypeStruct((_E, 48), jnp.float32),
            jax.ShapeDtypeStruct((_E, 48), jnp.float32),
            jax.ShapeDtypeStruct((_E, _D), jnp.float32),
            jax.ShapeDtypeStruct((_E, _D), jnp.float32),
        ],
    )(edge_attr, edge_sh, ew1, eb1, ew2, eb2, gw1, gb1, w2)


def _proj0_body(h_ref, w1_ref, gmid_ref, gbot_ref, tsrc_ref, tdst_ref):
    h = h_ref[...]
    hs = h[:, :_NS]
    tsrc_ref[:, :_D] = jnp.dot(h, w1_ref[...],
                               preferred_element_type=jnp.float32)
    tsrc_ref[:, _D:_TSP] = jnp.zeros((tsrc_ref.shape[0], _TSP - _D),
                                     jnp.float32)
    tsrc_ref[:, _D:_TS] = jnp.dot(hs, gmid_ref[...],
                                  preferred_element_type=jnp.float32)
    tdst_ref[...] = jnp.zeros(tdst_ref.shape, jnp.float32)
    tdst_ref[:, :48] = jnp.dot(hs, gbot_ref[...],
                               preferred_element_type=jnp.float32)


def _tc_proj0(h, w1, gmid, gbot):
    grid = (_N // _BN,)
    return pl.pallas_call(
        _proj0_body,
        grid=grid,
        in_specs=[
            pl.BlockSpec((_BN, _D), lambda i: (i, 0)),
            pl.BlockSpec((_D, _D), lambda i: (0, 0)),
            pl.BlockSpec((_NS, 48), lambda i: (0, 0)),
            pl.BlockSpec((_NS, 48), lambda i: (0, 0)),
        ],
        out_specs=[
            pl.BlockSpec((_BN, _TSP), lambda i: (i, 0)),
            pl.BlockSpec((_BN, _TDP), lambda i: (i, 0)),
        ],
        out_shape=[
            jax.ShapeDtypeStruct((_N, _TSP), jnp.float32),
            jax.ShapeDtypeStruct((_N, _TDP), jnp.float32),
        ],
    )(h, w1, gmid, gbot)


def _proj1_body(h_ref, p_ref, w1_ref, gmid_ref, gbot_ref,
                hout_ref, tsrc_ref, tdst_ref):
    h = h_ref[...] + p_ref[0] + p_ref[1]
    hout_ref[...] = h
    hs = h[:, :_NS]
    tsrc_ref[:, :_D] = jnp.dot(h, w1_ref[...],
                               preferred_element_type=jnp.float32)
    tsrc_ref[:, _D:_TSP] = jnp.zeros((tsrc_ref.shape[0], _TSP - _D),
                                     jnp.float32)
    tsrc_ref[:, _D:_TS] = jnp.dot(hs, gmid_ref[...],
                                  preferred_element_type=jnp.float32)
    tdst_ref[...] = jnp.zeros(tdst_ref.shape, jnp.float32)
    tdst_ref[:, :48] = jnp.dot(hs, gbot_ref[...],
                               preferred_element_type=jnp.float32)


def _tc_proj1(h, parts, w1, gmid, gbot):
    grid = (_N // _BN,)
    return pl.pallas_call(
        _proj1_body,
        grid=grid,
        in_specs=[
            pl.BlockSpec((_BN, _D), lambda i: (i, 0)),
            pl.BlockSpec((2, _BN, _D), lambda i: (0, i, 0)),
            pl.BlockSpec((_D, _D), lambda i: (0, 0)),
            pl.BlockSpec((_NS, 48), lambda i: (0, 0)),
            pl.BlockSpec((_NS, 48), lambda i: (0, 0)),
        ],
        out_specs=[
            pl.BlockSpec((_BN, _D), lambda i: (i, 0)),
            pl.BlockSpec((_BN, _TSP), lambda i: (i, 0)),
            pl.BlockSpec((_BN, _TDP), lambda i: (i, 0)),
        ],
        out_shape=[
            jax.ShapeDtypeStruct((_N, _D), jnp.float32),
            jax.ShapeDtypeStruct((_N, _TSP), jnp.float32),
            jax.ShapeDtypeStruct((_N, _TDP), jnp.float32),
        ],
    )(h, parts, w1, gmid, gbot)


def _edge_body(gsrc_ref, gdst_ref, pre_ref, shw_ref, gw2_ref, gb2_ref,
               msg_ref):
    gsrc = gsrc_ref[...]
    u = jnp.maximum(pre_ref[...] + gsrc[:, _D:_TS] + gdst_ref[:, :48], 0.0)
    gate = (jnp.dot(u, gw2_ref[...], preferred_element_type=jnp.float32)
            + gb2_ref[...])
    msg_ref[...] = gate * gsrc[:, :_D] * shw_ref[...]


def _tc_edge(gsrc, gdst, pre, shw, gw2, gb2):
    grid = (_E // _BE,)
    return pl.pallas_call(
        _edge_body,
        grid=grid,
        in_specs=[
            pl.BlockSpec((_BE, _TSP), lambda i: (i, 0)),
            pl.BlockSpec((_BE, _TDP), lambda i: (i, 0)),
            pl.BlockSpec((_BE, 48), lambda i: (i, 0)),
            pl.BlockSpec((_BE, _D), lambda i: (i, 0)),
            pl.BlockSpec((48, _D), lambda i: (0, 0)),
            pl.BlockSpec((1, _D), lambda i: (0, 0)),
        ],
        out_specs=pl.BlockSpec((_BE, _D), lambda i: (i, 0)),
        out_shape=jax.ShapeDtypeStruct((_E, _D), jnp.float32),
    )(gsrc, gdst, pre, shw, gw2, gb2)


def _final_body(h_ref, p_ref, o_ref):
    o_ref[...] = h_ref[...] + p_ref[0] + p_ref[1]


def _tc_final(h, parts):
    grid = (_N // _BN,)
    return pl.pallas_call(
        _final_body,
        grid=grid,
        in_specs=[
            pl.BlockSpec((_BN, _D), lambda i: (i, 0)),
            pl.BlockSpec((2, _BN, _D), lambda i: (0, i, 0)),
        ],
        out_specs=pl.BlockSpec((_BN, _D), lambda i: (i, 0)),
        out_shape=jax.ShapeDtypeStruct((_N, _D), jnp.float32),
    )(h, parts)


def kernel(x, edge_attr, edge_sh, emb_w1, emb_b1, emb_w2, emb_b2,
           gate_w1, gate_b1, gate_w2, gate_b2, W1, W2, edge_index):
    ei = edge_index.astype(jnp.int32)
    src2 = ei[0].reshape(_NCH, _CH)
    dst2 = ei[1].reshape(_NCH, _CH)

    pre0, pre1, shw0, shw1 = _tc_pre(
        edge_attr, edge_sh, emb_w1, emb_b1.reshape(1, _NS), emb_w2,
        emb_b2.reshape(1, _NS), gate_w1, gate_b1.reshape(2, 1, 48), W2)

    h = x
    parts = None
    for l in range(2):
        gmid = gate_w1[l, _NS:2 * _NS, :]
        gbot = gate_w1[l, 2 * _NS:3 * _NS, :]
        if l == 0:
            tsrc, tdst = _tc_proj0(h, W1[0], gmid, gbot)
        else:
            h, tsrc, tdst = _tc_proj1(h, parts, W1[1], gmid, gbot)
        gsrc, gdst = _sc_gather(tsrc, tdst, src2, dst2)
        msg = _tc_edge(gsrc, gdst,
                       pre0 if l == 0 else pre1,
                       shw0 if l == 0 else shw1,
                       gate_w2[l], gate_b2[l].reshape(1, _D))
        parts = _sc_scatter(msg, dst2)
    return _tc_final(h, parts)


# speedup vs baseline: 249.6676x; 1.1106x over previous
import functools

import jax
import jax.numpy as jnp
from jax import lax
from jax.experimental import pallas as pl
from jax.experimental.pallas import tpu as pltpu
from jax.experimental.pallas import tpu_sc as plsc

_NS = 16
_N = 10000
_E = 160000
_D = 128
_SH = 9
_DE = 64

_CH = 128
_NCH = _E // _CH
_NW = 32
_CPT = _NCH // _NW
_REM = _NCH - _CPT * _NW
_RPS = 624
_RTAIL = _N - 16 * _RPS

_mesh = plsc.VectorSubcoreMesh(core_axis_name="c", subcore_axis_name="s")


@functools.partial(
    pl.kernel,
    mesh=_mesh,
    out_type=jax.ShapeDtypeStruct((_E, 48), jnp.float32),
    scratch_types=[
        pltpu.VMEM((1, _CH), jnp.int32),
        pltpu.VMEM((1, _CH), jnp.int32),
        pltpu.VMEM((_CH, _D), jnp.float32),
        pltpu.VMEM((_CH, _D), jnp.float32),
        pltpu.VMEM((_CH, 48), jnp.float32),
        pltpu.SemaphoreType.DMA,
        pltpu.SemaphoreType.DMA,
    ],
)
def _sc_gather(tbl, src2, dst2, s_out,
               idx_s, idx_d, buf_s, buf_d, buf_u, sem_s, sem_d):
    wid = lax.axis_index("s") * 2 + lax.axis_index("c")
    c0 = wid * _CPT

    def do_chunk(ci):
        pltpu.sync_copy(src2.at[pl.ds(ci, 1)], idx_s)
        pltpu.sync_copy(dst2.at[pl.ds(ci, 1)], idx_d)
        cp_s = pltpu.async_copy(tbl.at[idx_s.at[0]], buf_s, sem_s)
        cp_d = pltpu.async_copy(tbl.at[idx_d.at[0]], buf_d, sem_d)
        cp_s.wait()
        cp_d.wait()

        def srow(r, carry):
            for k in range(3):
                buf_u[r, pl.ds(k * 16, 16)] = (
                    buf_s[r, pl.ds(k * 16, 16)]
                    + buf_d[r, pl.ds(48 + k * 16, 16)])
            return carry

        lax.fori_loop(0, _CH, srow, 0)
        pltpu.sync_copy(buf_u, s_out.at[pl.ds(ci * _CH, _CH)])

    def body(j, carry):
        do_chunk(c0 + j)
        return carry

    lax.fori_loop(0, _CPT, body, 0)

    @pl.when(wid < _REM)
    def _():
        do_chunk(_NW * _CPT + wid)


@functools.partial(
    pl.kernel,
    mesh=_mesh,
    out_type=jax.ShapeDtypeStruct((2, _N, _D), jnp.float32),
    scratch_types=[
        pltpu.VMEM((1, _CH), jnp.int32),
        pltpu.VMEM((1, _CH), jnp.int32),
        pltpu.VMEM((_CH, _D), jnp.float32),
        pltpu.VMEM((_CH, _D), jnp.float32),
        pltpu.VMEM_SHARED((_N, _D), jnp.float32),
        pltpu.SemaphoreType.DMA,
    ],
)
def _sc_scatter(gate2, a_tbl, src2, dst2, out, idx_s, idx_d, buf_g, buf_a,
                acc, sem_a):
    cid = lax.axis_index("c")
    sid = lax.axis_index("s")
    wid = sid * 2 + cid

    def zrow(i, carry):
        for k in range(_D // 16):
            buf_g[i, pl.ds(k * 16, 16)] = jnp.zeros((16,), jnp.float32)
        return carry

    lax.fori_loop(0, _CH, zrow, 0)
    r0 = sid * _RPS
    for t in range(4):
        pltpu.sync_copy(buf_g, acc.at[pl.ds(r0 + t * _CH, _CH)])
    pltpu.sync_copy(buf_g.at[pl.ds(0, _RPS - 4 * _CH)],
                    acc.at[pl.ds(r0 + 4 * _CH, _RPS - 4 * _CH)])

    @pl.when(sid == 0)
    def _():
        pltpu.sync_copy(buf_g.at[pl.ds(0, _RTAIL)],
                        acc.at[pl.ds(16 * _RPS, _RTAIL)])

    plsc.subcore_barrier()

    def do_chunk(ci):
        pltpu.sync_copy(src2.at[pl.ds(ci, 1)], idx_s)
        pltpu.sync_copy(dst2.at[pl.ds(ci, 1)], idx_d)
        cp_a = pltpu.async_copy(a_tbl.at[idx_s.at[0]], buf_a, sem_a)
        pltpu.sync_copy(gate2.at[pl.ds(ci * _CH, _CH)], buf_g)
        cp_a.wait()

        def mrow(r, carry):
            for k in range(_D // 16):
                sl = pl.ds(k * 16, 16)
                buf_g[r, sl] = buf_g[r, sl] * buf_a[r, sl]
            return carry

        lax.fori_loop(0, _CH, mrow, 0)
        pltpu.sync_copy(buf_g, acc.at[idx_d.at[0]], add=True)

    c0 = wid * _CPT

    def body(j, carry):
        do_chunk(c0 + j)
        return carry

    lax.fori_loop(0, _CPT, body, 0)

    @pl.when(wid < _REM)
    def _():
        do_chunk(_NW * _CPT + wid)

    plsc.subcore_barrier()
    pltpu.sync_copy(acc.at[pl.ds(r0, _RPS)], out.at[cid, pl.ds(r0, _RPS)])

    @pl.when(sid == 0)
    def _():
        pltpu.sync_copy(acc.at[pl.ds(16 * _RPS, _RTAIL)],
                        out.at[cid, pl.ds(16 * _RPS, _RTAIL)])


_BE = 2000
_BN = 1000


def _proj0_body(h_ref, w1_ref, gmid_ref, gbot_ref, a_ref, tbl_ref):
    h = h_ref[...]
    hs = h[:, :_NS]
    a_ref[...] = jnp.dot(h, w1_ref[...], preferred_element_type=jnp.float32)
    tbl_ref[:, :] = jnp.zeros(tbl_ref.shape, jnp.float32)
    tbl_ref[:, :48] = jnp.dot(hs, gmid_ref[...],
                              preferred_element_type=jnp.float32)
    tbl_ref[:, 48:96] = jnp.dot(hs, gbot_ref[...],
                                preferred_element_type=jnp.float32)


def _tc_proj0(h, w1, gmid, gbot):
    grid = (_N // _BN,)
    return pl.pallas_call(
        _proj0_body,
        grid=grid,
        in_specs=[
            pl.BlockSpec((_BN, _D), lambda i: (i, 0)),
            pl.BlockSpec((_D, _D), lambda i: (0, 0)),
            pl.BlockSpec((_NS, 48), lambda i: (0, 0)),
            pl.BlockSpec((_NS, 48), lambda i: (0, 0)),
        ],
        out_specs=[
            pl.BlockSpec((_BN, _D), lambda i: (i, 0)),
            pl.BlockSpec((_BN, _D), lambda i: (i, 0)),
        ],
        out_shape=[
            jax.ShapeDtypeStruct((_N, _D), jnp.float32),
            jax.ShapeDtypeStruct((_N, _D), jnp.float32),
        ],
    )(h, w1, gmid, gbot)


def _proj1_body(h_ref, p_ref, w1_ref, gmid_ref, gbot_ref,
                hout_ref, a_ref, tbl_ref):
    h = h_ref[...] + p_ref[0] + p_ref[1]
    hout_ref[...] = h
    hs = h[:, :_NS]
    a_ref[...] = jnp.dot(h, w1_ref[...], preferred_element_type=jnp.float32)
    tbl_ref[:, :] = jnp.zeros(tbl_ref.shape, jnp.float32)
    tbl_ref[:, :48] = jnp.dot(hs, gmid_ref[...],
                              preferred_element_type=jnp.float32)
    tbl_ref[:, 48:96] = jnp.dot(hs, gbot_ref[...],
                                preferred_element_type=jnp.float32)


def _tc_proj1(h, parts, w1, gmid, gbot):
    grid = (_N // _BN,)
    return pl.pallas_call(
        _proj1_body,
        grid=grid,
        in_specs=[
            pl.BlockSpec((_BN, _D), lambda i: (i, 0)),
            pl.BlockSpec((2, _BN, _D), lambda i: (0, i, 0)),
            pl.BlockSpec((_D, _D), lambda i: (0, 0)),
            pl.BlockSpec((_NS, 48), lambda i: (0, 0)),
            pl.BlockSpec((_NS, 48), lambda i: (0, 0)),
        ],
        out_specs=[
            pl.BlockSpec((_BN, _D), lambda i: (i, 0)),
            pl.BlockSpec((_BN, _D), lambda i: (i, 0)),
            pl.BlockSpec((_BN, _D), lambda i: (i, 0)),
        ],
        out_shape=[
            jax.ShapeDtypeStruct((_N, _D), jnp.float32),
            jax.ShapeDtypeStruct((_N, _D), jnp.float32),
            jax.ShapeDtypeStruct((_N, _D), jnp.float32),
        ],
    )(h, parts, w1, gmid, gbot)


def _edge_body(ea_ref, esh_ref, s_ref,
               ew1_ref, eb1_ref, ew2_ref, eb2_ref,
               gtop_ref, gb1_ref, gw2_ref, gb2_ref, w2_ref,
               gate2_ref):
    e = jnp.maximum(
        jnp.dot(ea_ref[...], ew1_ref[...], preferred_element_type=jnp.float32)
        + eb1_ref[...], 0.0)
    e = (jnp.dot(e, ew2_ref[...], preferred_element_type=jnp.float32)
         + eb2_ref[...])
    pre = (jnp.dot(e, gtop_ref[...], preferred_element_type=jnp.float32)
           + gb1_ref[...])
    u = jnp.maximum(pre + s_ref[...], 0.0)
    gate = (jnp.dot(u, gw2_ref[...], preferred_element_type=jnp.float32)
            + gb2_ref[...])
    shw = jnp.dot(esh_ref[...], w2_ref[...],
                  preferred_element_type=jnp.float32)
    gate2_ref[...] = gate * shw


def _tc_edge(edge_attr, edge_sh, s, ew1, eb1, ew2, eb2,
             gtop, gb1, gw2, gb2, w2l):
    grid = (_E // _BE,)
    return pl.pallas_call(
        _edge_body,
        grid=grid,
        in_specs=[
            pl.BlockSpec((_BE, _DE), lambda i: (i, 0)),
            pl.BlockSpec((_BE, _SH), lambda i: (i, 0)),
            pl.BlockSpec((_BE, 48), lambda i: (i, 0)),
            pl.BlockSpec((_DE, _NS), lambda i: (0, 0)),
            pl.BlockSpec((1, _NS), lambda i: (0, 0)),
            pl.BlockSpec((_NS, _NS), lambda i: (0, 0)),
            pl.BlockSpec((1, _NS), lambda i: (0, 0)),
            pl.BlockSpec((_NS, 48), lambda i: (0, 0)),
            pl.BlockSpec((1, 48), lambda i: (0, 0)),
            pl.BlockSpec((48, _D), lambda i: (0, 0)),
            pl.BlockSpec((1, _D), lambda i: (0, 0)),
            pl.BlockSpec((_SH, _D), lambda i: (0, 0)),
        ],
        out_specs=pl.BlockSpec((_BE, _D), lambda i: (i, 0)),
        out_shape=jax.ShapeDtypeStruct((_E, _D), jnp.float32),
    )(edge_attr, edge_sh, s, ew1, eb1, ew2, eb2, gtop, gb1, gw2, gb2, w2l)


def _final_body(h_ref, p_ref, o_ref):
    o_ref[...] = h_ref[...] + p_ref[0] + p_ref[1]


def _tc_final(h, parts):
    grid = (_N // _BN,)
    return pl.pallas_call(
        _final_body,
        grid=grid,
        in_specs=[
            pl.BlockSpec((_BN, _D), lambda i: (i, 0)),
            pl.BlockSpec((2, _BN, _D), lambda i: (0, i, 0)),
        ],
        out_specs=pl.BlockSpec((_BN, _D), lambda i: (i, 0)),
        out_shape=jax.ShapeDtypeStruct((_N, _D), jnp.float32),
    )(h, parts)


def kernel(x, edge_attr, edge_sh, emb_w1, emb_b1, emb_w2, emb_b2,
           gate_w1, gate_b1, gate_w2, gate_b2, W1, W2, edge_index):
    ei = edge_index.astype(jnp.int32)
    src2 = ei[0].reshape(_NCH, _CH)
    dst2 = ei[1].reshape(_NCH, _CH)

    eb1 = emb_b1.reshape(1, _NS)
    eb2 = emb_b2.reshape(1, _NS)

    h = x
    parts = None
    for l in range(2):
        gtop = gate_w1[l, :_NS, :]
        gmid = gate_w1[l, _NS:2 * _NS, :]
        gbot = gate_w1[l, 2 * _NS:3 * _NS, :]
        if l == 0:
            a, tbl = _tc_proj0(h, W1[0], gmid, gbot)
        else:
            h, a, tbl = _tc_proj1(h, parts, W1[1], gmid, gbot)
        s = _sc_gather(tbl, src2, dst2)
        gate2 = _tc_edge(edge_attr, edge_sh, s, emb_w1, eb1, emb_w2, eb2,
                         gtop, gate_b1[l].reshape(1, 48),
                         gate_w2[l], gate_b2[l].reshape(1, _D), W2[l])
        parts = _sc_scatter(gate2, a, src2, dst2)
    return _tc_final(h, parts)


# speedup vs baseline: 310.6751x; 1.2444x over previous
import functools

import jax
import jax.numpy as jnp
from jax import lax
from jax.experimental import pallas as pl
from jax.experimental.pallas import tpu as pltpu
from jax.experimental.pallas import tpu_sc as plsc

_NS = 16
_N = 10000
_E = 160000
_D = 128
_SH = 9
_DE = 64

_CH = 128
_NCH = _E // _CH
_NW = 32
_NSL = 40
_RPS = 624
_RTAIL = _N - 16 * _RPS

_mesh = plsc.VectorSubcoreMesh(core_axis_name="c", subcore_axis_name="s")


def _slot_valid(c0, t):
    return ((c0 + t) < _NCH) & (t < _NSL)


def _preload_idx(src2, dst2, idx_s, idx_d, c0):
    pltpu.sync_copy(src2.at[pl.ds(c0, _NSL)], idx_s)
    pltpu.sync_copy(dst2.at[pl.ds(c0, _NSL)], idx_d)


@functools.partial(
    pl.kernel,
    mesh=_mesh,
    out_type=jax.ShapeDtypeStruct((_E, 48), jnp.float32),
    scratch_types=[
        pltpu.VMEM((_NSL, _CH), jnp.int32),
        pltpu.VMEM((_NSL, _CH), jnp.int32),
        pltpu.VMEM((_CH, _D), jnp.float32),
        pltpu.VMEM((_CH, _D), jnp.float32),
        pltpu.VMEM((_CH, _D), jnp.float32),
        pltpu.VMEM((_CH, _D), jnp.float32),
        pltpu.VMEM((_CH, 48), jnp.float32),
        pltpu.VMEM((_CH, 48), jnp.float32),
        pltpu.SemaphoreType.DMA,
        pltpu.SemaphoreType.DMA,
        pltpu.SemaphoreType.DMA,
        pltpu.SemaphoreType.DMA,
    ],
)
def _sc_gather(tbl, src2, dst2, s_out, idx_s, idx_d,
               buf_s0, buf_s1, buf_d0, buf_d1, buf_u0, buf_u1,
               sem_s0, sem_s1, sem_d0, sem_d1):
    wid = lax.axis_index("s") * 2 + lax.axis_index("c")
    c0 = wid * _NSL
    _preload_idx(src2, dst2, idx_s, idx_d, c0)

    def start(t, buf_s, buf_d, sem_s, sem_d):
        pltpu.async_copy(tbl.at[idx_s.at[t]], buf_s, sem_s)
        pltpu.async_copy(tbl.at[idx_d.at[t]], buf_d, sem_d)

    def wait(buf_s, buf_d, sem_s, sem_d):
        pltpu.make_async_copy(tbl.at[pl.ds(0, _CH)], buf_s, sem_s).wait()
        pltpu.make_async_copy(tbl.at[pl.ds(0, _CH)], buf_d, sem_d).wait()

    def compute_store(t, buf_s, buf_d, buf_u):
        def srow(r2, carry):
            for dr in range(2):
                r = 2 * r2 + dr
                for k in range(3):
                    buf_u[r, pl.ds(k * 16, 16)] = (
                        buf_s[r, pl.ds(k * 16, 16)]
                        + buf_d[r, pl.ds(48 + k * 16, 16)])
            return carry

        lax.fori_loop(0, _CH // 2, srow, 0)
        pltpu.sync_copy(buf_u, s_out.at[pl.ds((c0 + t) * _CH, _CH)])

    start(0, buf_s0, buf_d0, sem_s0, sem_d0)

    def body(g, carry):
        t0 = 2 * g
        t1 = t0 + 1

        @pl.when(_slot_valid(c0, t1))
        def _():
            start(t1, buf_s1, buf_d1, sem_s1, sem_d1)

        @pl.when(_slot_valid(c0, t0))
        def _():
            wait(buf_s0, buf_d0, sem_s0, sem_d0)
            compute_store(t0, buf_s0, buf_d0, buf_u0)

        @pl.when(_slot_valid(c0, t0 + 2))
        def _():
            start(t0 + 2, buf_s0, buf_d0, sem_s0, sem_d0)

        @pl.when(_slot_valid(c0, t1))
        def _():
            wait(buf_s1, buf_d1, sem_s1, sem_d1)
            compute_store(t1, buf_s1, buf_d1, buf_u1)

        return carry

    lax.fori_loop(0, _NSL // 2, body, 0)


@functools.partial(
    pl.kernel,
    mesh=_mesh,
    out_type=jax.ShapeDtypeStruct((2, _N, _D), jnp.float32),
    scratch_types=[
        pltpu.VMEM((_NSL, _CH), jnp.int32),
        pltpu.VMEM((_NSL, _CH), jnp.int32),
        pltpu.VMEM((_CH, _D), jnp.float32),
        pltpu.VMEM((_CH, _D), jnp.float32),
        pltpu.VMEM_SHARED((_N, _D), jnp.float32),
        pltpu.SemaphoreType.DMA,
        pltpu.SemaphoreType.DMA,
    ],
)
def _sc_scatter(gate2, a_tbl, src2, dst2, out, idx_s, idx_d,
                buf_g0, buf_a0, acc, sem_g0, sem_a0):
    cid = lax.axis_index("c")
    sid = lax.axis_index("s")
    wid = sid * 2 + cid
    c0 = wid * _NSL
    _preload_idx(src2, dst2, idx_s, idx_d, c0)

    def zrow(i, carry):
        for k in range(_D // 16):
            buf_g0[i, pl.ds(k * 16, 16)] = jnp.zeros((16,), jnp.float32)
        return carry

    lax.fori_loop(0, _CH, zrow, 0)
    r0 = sid * _RPS
    for t in range(4):
        pltpu.sync_copy(buf_g0, acc.at[pl.ds(r0 + t * _CH, _CH)])
    pltpu.sync_copy(buf_g0.at[pl.ds(0, _RPS - 4 * _CH)],
                    acc.at[pl.ds(r0 + 4 * _CH, _RPS - 4 * _CH)])

    @pl.when(sid == 0)
    def _():
        pltpu.sync_copy(buf_g0.at[pl.ds(0, _RTAIL)],
                        acc.at[pl.ds(16 * _RPS, _RTAIL)])

    plsc.subcore_barrier()

    pltpu.async_copy(a_tbl.at[idx_s.at[0]], buf_a0, sem_a0)
    pltpu.async_copy(gate2.at[pl.ds(c0 * _CH, _CH)], buf_g0, sem_g0)

    def body(t, carry):
        @pl.when(_slot_valid(c0, t))
        def _():
            pltpu.make_async_copy(a_tbl.at[pl.ds(0, _CH)], buf_a0,
                                  sem_a0).wait()
            pltpu.make_async_copy(gate2.at[pl.ds(0, _CH)], buf_g0,
                                  sem_g0).wait()

            def mrow(r2, c):
                for dr in range(2):
                    r = 2 * r2 + dr
                    for k in range(_D // 16):
                        sl = pl.ds(k * 16, 16)
                        buf_g0[r, sl] = buf_g0[r, sl] * buf_a0[r, sl]
                return c

            lax.fori_loop(0, _CH // 2, mrow, 0)

            @pl.when(_slot_valid(c0, t + 1))
            def _():
                pltpu.async_copy(a_tbl.at[idx_s.at[t + 1]], buf_a0, sem_a0)

            pltpu.sync_copy(buf_g0, acc.at[idx_d.at[t]], add=True)

            @pl.when(_slot_valid(c0, t + 1))
            def _():
                pltpu.async_copy(gate2.at[pl.ds((c0 + t + 1) * _CH, _CH)],
                                 buf_g0, sem_g0)

        return carry

    lax.fori_loop(0, _NSL, body, 0)

    plsc.subcore_barrier()
    pltpu.sync_copy(acc.at[pl.ds(r0, _RPS)], out.at[cid, pl.ds(r0, _RPS)])

    @pl.when(sid == 0)
    def _():
        pltpu.sync_copy(acc.at[pl.ds(16 * _RPS, _RTAIL)],
                        out.at[cid, pl.ds(16 * _RPS, _RTAIL)])


_BE = 2000
_BN = 1000


def _proj0_body(h_ref, w1_ref, gmid_ref, gbot_ref, a_ref, tbl_ref):
    h = h_ref[...]
    hs = h[:, :_NS]
    a_ref[...] = jnp.dot(h, w1_ref[...], preferred_element_type=jnp.float32)
    tbl_ref[:, :] = jnp.zeros(tbl_ref.shape, jnp.float32)
    tbl_ref[:, :48] = jnp.dot(hs, gmid_ref[...],
                              preferred_element_type=jnp.float32)
    tbl_ref[:, 48:96] = jnp.dot(hs, gbot_ref[...],
                                preferred_element_type=jnp.float32)


def _tc_proj0(h, w1, gmid, gbot):
    grid = (_N // _BN,)
    return pl.pallas_call(
        _proj0_body,
        grid=grid,
        in_specs=[
            pl.BlockSpec((_BN, _D), lambda i: (i, 0)),
            pl.BlockSpec((_D, _D), lambda i: (0, 0)),
            pl.BlockSpec((_NS, 48), lambda i: (0, 0)),
            pl.BlockSpec((_NS, 48), lambda i: (0, 0)),
        ],
        out_specs=[
            pl.BlockSpec((_BN, _D), lambda i: (i, 0)),
            pl.BlockSpec((_BN, _D), lambda i: (i, 0)),
        ],
        out_shape=[
            jax.ShapeDtypeStruct((_N, _D), jnp.float32),
            jax.ShapeDtypeStruct((_N, _D), jnp.float32),
        ],
    )(h, w1, gmid, gbot)


def _proj1_body(h_ref, p_ref, w1_ref, gmid_ref, gbot_ref,
                hout_ref, a_ref, tbl_ref):
    h = h_ref[...] + p_ref[0] + p_ref[1]
    hout_ref[...] = h
    hs = h[:, :_NS]
    a_ref[...] = jnp.dot(h, w1_ref[...], preferred_element_type=jnp.float32)
    tbl_ref[:, :] = jnp.zeros(tbl_ref.shape, jnp.float32)
    tbl_ref[:, :48] = jnp.dot(hs, gmid_ref[...],
                              preferred_element_type=jnp.float32)
    tbl_ref[:, 48:96] = jnp.dot(hs, gbot_ref[...],
                                preferred_element_type=jnp.float32)


def _tc_proj1(h, parts, w1, gmid, gbot):
    grid = (_N // _BN,)
    return pl.pallas_call(
        _proj1_body,
        grid=grid,
        in_specs=[
            pl.BlockSpec((_BN, _D), lambda i: (i, 0)),
            pl.BlockSpec((2, _BN, _D), lambda i: (0, i, 0)),
            pl.BlockSpec((_D, _D), lambda i: (0, 0)),
            pl.BlockSpec((_NS, 48), lambda i: (0, 0)),
            pl.BlockSpec((_NS, 48), lambda i: (0, 0)),
        ],
        out_specs=[
            pl.BlockSpec((_BN, _D), lambda i: (i, 0)),
            pl.BlockSpec((_BN, _D), lambda i: (i, 0)),
            pl.BlockSpec((_BN, _D), lambda i: (i, 0)),
        ],
        out_shape=[
            jax.ShapeDtypeStruct((_N, _D), jnp.float32),
            jax.ShapeDtypeStruct((_N, _D), jnp.float32),
            jax.ShapeDtypeStruct((_N, _D), jnp.float32),
        ],
    )(h, parts, w1, gmid, gbot)


def _edge_body(ea_ref, esh_ref, s_ref,
               ew1_ref, eb1_ref, ew2_ref, eb2_ref,
               gtop_ref, gb1_ref, gw2_ref, gb2_ref, w2_ref,
               gate2_ref):
    e = jnp.maximum(
        jnp.dot(ea_ref[...], ew1_ref[...], preferred_element_type=jnp.float32)
        + eb1_ref[...], 0.0)
    e = (jnp.dot(e, ew2_ref[...], preferred_element_type=jnp.float32)
         + eb2_ref[...])
    pre = (jnp.dot(e, gtop_ref[...], preferred_element_type=jnp.float32)
           + gb1_ref[...])
    u = jnp.maximum(pre + s_ref[...], 0.0)
    gate = (jnp.dot(u, gw2_ref[...], preferred_element_type=jnp.float32)
            + gb2_ref[...])
    shw = jnp.dot(esh_ref[...], w2_ref[...],
                  preferred_element_type=jnp.float32)
    gate2_ref[...] = gate * shw


def _tc_edge(edge_attr, edge_sh, s, ew1, eb1, ew2, eb2,
             gtop, gb1, gw2, gb2, w2l):
    grid = (_E // _BE,)
    return pl.pallas_call(
        _edge_body,
        grid=grid,
        in_specs=[
            pl.BlockSpec((_BE, _DE), lambda i: (i, 0)),
            pl.BlockSpec((_BE, _SH), lambda i: (i, 0)),
            pl.BlockSpec((_BE, 48), lambda i: (i, 0)),
            pl.BlockSpec((_DE, _NS), lambda i: (0, 0)),
            pl.BlockSpec((1, _NS), lambda i: (0, 0)),
            pl.BlockSpec((_NS, _NS), lambda i: (0, 0)),
            pl.BlockSpec((1, _NS), lambda i: (0, 0)),
            pl.BlockSpec((_NS, 48), lambda i: (0, 0)),
            pl.BlockSpec((1, 48), lambda i: (0, 0)),
            pl.BlockSpec((48, _D), lambda i: (0, 0)),
            pl.BlockSpec((1, _D), lambda i: (0, 0)),
            pl.BlockSpec((_SH, _D), lambda i: (0, 0)),
        ],
        out_specs=pl.BlockSpec((_BE, _D), lambda i: (i, 0)),
        out_shape=jax.ShapeDtypeStruct((_E, _D), jnp.float32),
    )(edge_attr, edge_sh, s, ew1, eb1, ew2, eb2, gtop, gb1, gw2, gb2, w2l)


def _final_body(h_ref, p_ref, o_ref):
    o_ref[...] = h_ref[...] + p_ref[0] + p_ref[1]


def _tc_final(h, parts):
    grid = (_N // _BN,)
    return pl.pallas_call(
        _final_body,
        grid=grid,
        in_specs=[
            pl.BlockSpec((_BN, _D), lambda i: (i, 0)),
            pl.BlockSpec((2, _BN, _D), lambda i: (0, i, 0)),
        ],
        out_specs=pl.BlockSpec((_BN, _D), lambda i: (i, 0)),
        out_shape=jax.ShapeDtypeStruct((_N, _D), jnp.float32),
    )(h, parts)


def kernel(x, edge_attr, edge_sh, emb_w1, emb_b1, emb_w2, emb_b2,
           gate_w1, gate_b1, gate_w2, gate_b2, W1, W2, edge_index):
    ei = edge_index.astype(jnp.int32)
    pad = _NW * _NSL - _NCH
    src2 = jnp.pad(ei[0].reshape(_NCH, _CH), ((0, pad), (0, 0)))
    dst2 = jnp.pad(ei[1].reshape(_NCH, _CH), ((0, pad), (0, 0)))

    eb1 = emb_b1.reshape(1, _NS)
    eb2 = emb_b2.reshape(1, _NS)

    h = x
    parts = None
    for l in range(2):
        gtop = gate_w1[l, :_NS, :]
        gmid = gate_w1[l, _NS:2 * _NS, :]
        gbot = gate_w1[l, 2 * _NS:3 * _NS, :]
        if l == 0:
            a, tbl = _tc_proj0(h, W1[0], gmid, gbot)
        else:
            h, a, tbl = _tc_proj1(h, parts, W1[1], gmid, gbot)
        s = _sc_gather(tbl, src2, dst2)
        gate2 = _tc_edge(edge_attr, edge_sh, s, emb_w1, eb1, emb_w2, eb2,
                         gtop, gate_b1[l].reshape(1, 48),
                         gate_w2[l], gate_b2[l].reshape(1, _D), W2[l])
        parts = _sc_scatter(gate2, a, src2, dst2)
    return _tc_final(h, parts)
